# Initial kernel scaffold; baseline (speedup 1.0000x reference)
import jax
import jax.numpy as jnp
from jax.experimental import pallas as pl


def kernel(x, edge_index, lin1_W, lin1_b, Ws, Wn, bn, cls_W1, cls_b1, cls_W2, cls_b2):
    raise NotImplementedError("write your pallas kernel here")


# speedup vs baseline: 3.6441x; 3.6441x over previous
import functools

import jax
import jax.numpy as jnp
from jax import lax
from jax.experimental import pallas as pl
from jax.experimental.pallas import tpu as pltpu
from jax.experimental.pallas import tpu_sc as plsc

_NC = 2
_NS = 16
_NW = _NC * _NS
_K = 128


def _sc_mesh():
    return plsc.VectorSubcoreMesh(
        core_axis_name="c", subcore_axis_name="s",
        num_cores=_NC, num_subcores=_NS)


def _fill_rows(ref, nrows, hp, value):
    vec = jnp.full((16,), value, jnp.float32)

    def fill_row(i, carry):
        for c in range(hp // 16):
            ref[i, pl.ds(16 * c, 16)] = vec
        return carry

    lax.fori_loop(0, nrows, fill_row, 0)


def _make_sc_segment_sum(hp, n_pad, e_pad):
    chunks = e_pad // (_NW * _K)
    rpt = n_pad // _NS
    zr = 64

    @functools.partial(
        pl.kernel,
        out_type=jax.ShapeDtypeStruct((_NC * n_pad, hp), jnp.float32),
        mesh=_sc_mesh(),
        scratch_types=[
            pltpu.VMEM((_K,), jnp.int32),
            pltpu.VMEM((_K,), jnp.int32),
            pltpu.VMEM((_K, hp), jnp.float32),
            pltpu.VMEM((zr, hp), jnp.float32),
            pltpu.VMEM_SHARED((n_pad, hp), jnp.float32),
            pltpu.SemaphoreType.DMA,
        ],
    )
    def sc_kernel(src_hbm, dst_hbm, h_hbm, out_hbm, sidx, didx, rows, zbuf,
                  agg, sem):
        cid = lax.axis_index("c")
        sid = lax.axis_index("s")
        wid = sid * _NC + cid
        _fill_rows(zbuf, zr, hp, 0.0)

        base_r = sid * rpt

        def zcopy(i, carry):
            pltpu.sync_copy(zbuf, agg.at[pl.ds(base_r + i * zr, zr)])
            return carry

        lax.fori_loop(0, rpt // zr, zcopy, 0)
        plsc.subcore_barrier()

        def body(i, carry):
            base = (wid * chunks + i) * _K
            pltpu.sync_copy(src_hbm.at[pl.ds(base, _K)], sidx)
            pltpu.sync_copy(dst_hbm.at[pl.ds(base, _K)], didx)
            pltpu.async_copy(h_hbm.at[sidx], rows, sem).wait()
            pltpu.sync_copy(rows, agg.at[didx], add=True)
            return carry

        lax.fori_loop(0, chunks, body, 0)
        plsc.subcore_barrier()

        pltpu.sync_copy(
            agg.at[pl.ds(base_r, rpt)],
            out_hbm.at[pl.ds(cid * n_pad + base_r, rpt)])

    return sc_kernel


def _make_sc_degree(hp, n_pad, e_pad):
    chunks = e_pad // (_NW * _K)
    rpt = n_pad // _NS
    zr = 64

    @functools.partial(
        pl.kernel,
        out_type=jax.ShapeDtypeStruct((_NC * n_pad, hp), jnp.float32),
        mesh=_sc_mesh(),
        scratch_types=[
            pltpu.VMEM((_K,), jnp.int32),
            pltpu.VMEM((_K, hp), jnp.float32),
            pltpu.VMEM((zr, hp), jnp.float32),
            pltpu.VMEM_SHARED((n_pad, hp), jnp.float32),
        ],
    )
    def sc_kernel(dst_hbm, out_hbm, didx, ones, zbuf, cnt):
        cid = lax.axis_index("c")
        sid = lax.axis_index("s")
        wid = sid * _NC + cid
        _fill_rows(zbuf, zr, hp, 0.0)
        _fill_rows(ones, _K, hp, 1.0)

        base_r = sid * rpt

        def zcopy(i, carry):
            pltpu.sync_copy(zbuf, cnt.at[pl.ds(base_r + i * zr, zr)])
            return carry

        lax.fori_loop(0, rpt // zr, zcopy, 0)
        plsc.subcore_barrier()

        def body(i, carry):
            base = (wid * chunks + i) * _K
            pltpu.sync_copy(dst_hbm.at[pl.ds(base, _K)], didx)
            pltpu.sync_copy(ones, cnt.at[didx], add=True)
            return carry

        lax.fori_loop(0, chunks, body, 0)
        plsc.subcore_barrier()

        pltpu.sync_copy(
            cnt.at[pl.ds(base_r, rpt)],
            out_hbm.at[pl.ds(cid * n_pad + base_r, rpt)])

    return sc_kernel


def kernel(x, edge_index, lin1_W, lin1_b, Ws, Wn, bn,
           cls_W1, cls_b1, cls_W2, cls_b2):
    n, d = x.shape
    h = lin1_W.shape[1]
    num_layers = Ws.shape[0]
    c_out = cls_W2.shape[1]
    e = edge_index.shape[1]

    n_pad = ((n // (_NS * 16)) + 1) * (_NS * 16)
    chunks = -(-e // (_NW * _K))
    e_pad = _NW * _K * chunks
    r = 1000
    grid_n = n // r

    src = edge_index[0].astype(jnp.int32)
    dst = edge_index[1].astype(jnp.int32)
    pad = e_pad - e
    src_p = jnp.concatenate([src, jnp.zeros((pad,), jnp.int32)])
    dst_p = jnp.concatenate([dst, jnp.full((pad,), n, jnp.int32)])

    def lin1_body(x_ref, w_ref, b_ref, out_ref):
        out_ref[...] = jnp.dot(x_ref[...], w_ref[...],
                               preferred_element_type=jnp.float32) + b_ref[...]

    h0 = pl.pallas_call(
        lin1_body,
        grid=(grid_n,),
        in_specs=[
            pl.BlockSpec((r, d), lambda i: (i, 0)),
            pl.BlockSpec((d, h), lambda i: (0, 0)),
            pl.BlockSpec((1, h), lambda i: (0, 0)),
        ],
        out_specs=pl.BlockSpec((r, h), lambda i: (i, 0)),
        out_shape=jax.ShapeDtypeStruct((n, h), jnp.float32),
    )(x, lin1_W, lin1_b.reshape(1, h))

    sc_plain = _make_sc_segment_sum(h, n_pad, e_pad)
    sc_deg = _make_sc_degree(h, n_pad, e_pad)

    pdeg = sc_deg(dst_p).reshape(_NC, n_pad, h)
    p1 = sc_plain(src_p, dst_p, h0).reshape(_NC, n_pad, h)

    def sage1_body(h_ref, p_ref, pdeg_ref, ws_ref, wn_ref, bn_ref,
                   h_out, deg_out):
        hv = h_ref[...]
        p = p_ref[...]
        agg = p[0] + p[1]
        pd = pdeg_ref[...]
        deg = jnp.maximum(pd[0][:, :1] + pd[1][:, :1], 1.0)
        mean = agg / deg
        hn = jax.nn.relu(
            jnp.dot(hv, ws_ref[...], preferred_element_type=jnp.float32)
            + jnp.dot(mean, wn_ref[...], preferred_element_type=jnp.float32)
            + bn_ref[...])
        h_out[...] = hn
        deg_out[...] = jnp.broadcast_to(deg, (r, h))

    h1, deg = pl.pallas_call(
        sage1_body,
        grid=(grid_n,),
        in_specs=[
            pl.BlockSpec((r, h), lambda i: (i, 0)),
            pl.BlockSpec((_NC, r, h), lambda i: (0, i, 0)),
            pl.BlockSpec((_NC, r, h), lambda i: (0, i, 0)),
            pl.BlockSpec((h, h), lambda i: (0, 0)),
            pl.BlockSpec((h, h), lambda i: (0, 0)),
            pl.BlockSpec((1, h), lambda i: (0, 0)),
        ],
        out_specs=[
            pl.BlockSpec((r, h), lambda i: (i, 0)),
            pl.BlockSpec((r, h), lambda i: (i, 0)),
        ],
        out_shape=[
            jax.ShapeDtypeStruct((n, h), jnp.float32),
            jax.ShapeDtypeStruct((n, h), jnp.float32),
        ],
    )(h0, p1, pdeg, Ws[0], Wn[0], bn[0].reshape(1, h))

    p2 = sc_plain(src_p, dst_p, h1).reshape(_NC, n_pad, h)

    def sage2_body(h_ref, p_ref, deg_ref, ws_ref, wn_ref, bn_ref,
                   h_out, x5_out):
        p = p_ref[...]
        mean = (p[0] + p[1]) / deg_ref[...]
        hv = h_ref[...]
        hn = jax.nn.relu(
            jnp.dot(hv, ws_ref[...], preferred_element_type=jnp.float32)
            + jnp.dot(mean, wn_ref[...], preferred_element_type=jnp.float32)
            + bn_ref[...])
        h_out[...] = hn
        x5_out[...] = jnp.maximum(hv, hn)

    h2, x5 = pl.pallas_call(
        sage2_body,
        grid=(grid_n,),
        in_specs=[
            pl.BlockSpec((r, h), lambda i: (i, 0)),
            pl.BlockSpec((_NC, r, h), lambda i: (0, i, 0)),
            pl.BlockSpec((r, h), lambda i: (i, 0)),
            pl.BlockSpec((h, h), lambda i: (0, 0)),
            pl.BlockSpec((h, h), lambda i: (0, 0)),
            pl.BlockSpec((1, h), lambda i: (0, 0)),
        ],
        out_specs=[
            pl.BlockSpec((r, h), lambda i: (i, 0)),
            pl.BlockSpec((r, h), lambda i: (i, 0)),
        ],
        out_shape=[
            jax.ShapeDtypeStruct((n, h), jnp.float32),
            jax.ShapeDtypeStruct((n, h), jnp.float32),
        ],
    )(h1, p2, deg, Ws[1], Wn[1], bn[1].reshape(1, h))

    p3 = sc_plain(src_p, dst_p, h2).reshape(_NC, n_pad, h)

    w2p = jnp.pad(cls_W2, ((0, 0), (0, h - c_out)))
    b2p = jnp.pad(cls_b2, (0, h - c_out)).reshape(1, h)

    def sage3_body(h_ref, x5_ref, p_ref, deg_ref, ws_ref, wn_ref, bn_ref,
                   w1_ref, b1_ref, w2_ref, b2_ref, out_ref):
        p = p_ref[...]
        mean = (p[0] + p[1]) / deg_ref[...]
        hv = h_ref[...]
        hn = jax.nn.relu(
            jnp.dot(hv, ws_ref[...], preferred_element_type=jnp.float32)
            + jnp.dot(mean, wn_ref[...], preferred_element_type=jnp.float32)
            + bn_ref[...])
        x5v = jnp.maximum(x5_ref[...], hn)
        hid = jax.nn.relu(
            jnp.dot(x5v, w1_ref[...], preferred_element_type=jnp.float32)
            + b1_ref[...])
        out_ref[...] = jnp.dot(
            hid, w2_ref[...], preferred_element_type=jnp.float32) + b2_ref[...]

    logits_pad = pl.pallas_call(
        sage3_body,
        grid=(grid_n,),
        in_specs=[
            pl.BlockSpec((r, h), lambda i: (i, 0)),
            pl.BlockSpec((r, h), lambda i: (i, 0)),
            pl.BlockSpec((_NC, r, h), lambda i: (0, i, 0)),
            pl.BlockSpec((r, h), lambda i: (i, 0)),
            pl.BlockSpec((h, h), lambda i: (0, 0)),
            pl.BlockSpec((h, h), lambda i: (0, 0)),
            pl.BlockSpec((1, h), lambda i: (0, 0)),
            pl.BlockSpec((h, h), lambda i: (0, 0)),
            pl.BlockSpec((1, h), lambda i: (0, 0)),
            pl.BlockSpec((h, h), lambda i: (0, 0)),
            pl.BlockSpec((1, h), lambda i: (0, 0)),
        ],
        out_specs=pl.BlockSpec((r, h), lambda i: (i, 0)),
        out_shape=jax.ShapeDtypeStruct((n, h), jnp.float32),
    )(h2, x5, p3, deg, Ws[2], Wn[2], bn[2].reshape(1, h),
      cls_W1, cls_b1.reshape(1, h), w2p, b2p)

    return logits_pad[:, :c_out]
